# baseline (speedup 1.0000x reference)
import jax
import jax.numpy as jnp
from jax.experimental import pallas as pl
from jax.experimental.pallas import tpu as pltpu

_BB = 8
_P = 8


def _fused_kernel(ux_ref, uy_ref, bbox_ref, wp_ref, img_ref, wf_ref, bf_ref,
                  wc_ref, bc_ref, wg_ref, bg_ref,
                  poly_ref, cls_ref, pred_ref, orig_ref, binh_ref, feat_ref):
    BB, C, H, W = img_ref.shape
    P = ux_ref.shape[1]
    Hs, Ws = H // 2, W // 2

    x = img_ref[...]

    poly_ref[...] = (wp_ref[0] * x[:, 0:1] + wp_ref[1] * x[:, 1:2]
                     + wp_ref[2] * x[:, 2:3] + wp_ref[3])

    pooled = jnp.mean(x, axis=(2, 3))
    logits = (pooled[:, 0:1] * wc_ref[0:1, :]
              + pooled[:, 1:2] * wc_ref[1:2, :]
              + pooled[:, 2:3] * wc_ref[2:3, :] + bc_ref[...])
    cls_ref[...] = jax.nn.sigmoid(logits)

    bw = bbox_ref[:, 2:3]
    bh = bbox_ref[:, 3:4]
    hx = jnp.floor(ux_ref[...] * bw)
    hy = jnp.floor(uy_ref[...] * bh)
    wg_ = jnp.maximum(bw, 1e-6)
    hg_ = jnp.maximum(bh, 1e-6)
    orig_ref[...] = jnp.concatenate(
        [hy[:, :, None], hx[:, :, None]], axis=-1).astype(jnp.int32)
    by = hy / hg_
    bx = hx / wg_
    binh = jnp.concatenate([by[:, :, None], bx[:, :, None]], axis=-1)
    binh_ref[...] = binh
    fy = jnp.floor(hy * Hs / hg_).astype(jnp.int32)
    fx = jnp.floor(hx * Ws / wg_).astype(jnp.int32)
    feat_ref[...] = jnp.concatenate(
        [fy[:, :, None], fx[:, :, None]], axis=-1)

    row = 2 * jnp.clip(fy, 0, Hs - 1)
    col = 2 * jnp.clip(fx, 0, Ws - 1)
    riota = jax.lax.broadcasted_iota(jnp.int32, (BB, P, H, W), 2)
    ciota = jax.lax.broadcasted_iota(jnp.int32, (BB, P, H, W), 3)
    sel = ((riota == row[:, :, None, None])
           & (ciota == col[:, :, None, None])).astype(jnp.float32)
    g0 = jnp.sum(sel * x[:, 0:1], axis=(2, 3))
    g1 = jnp.sum(sel * x[:, 1:2], axis=(2, 3))
    g2 = jnp.sum(sel * x[:, 2:3], axis=(2, 3))

    wcomb = jnp.dot(wf_ref[...], wg_ref[...],
                    preferred_element_type=jnp.float32)
    const = jnp.dot(bf_ref[...], wg_ref[...],
                    preferred_element_type=jnp.float32) + bg_ref[...]
    offs = (g0[:, :, None] * wcomb[0:1, :]
            + g1[:, :, None] * wcomb[1:2, :]
            + g2[:, :, None] * wcomb[2:3, :] + const)
    pred_ref[...] = binh + offs


def kernel(img, bbox, w_poly, b_poly, w_feat, b_feat, w_cls, b_cls, w_gcn, b_gcn):
    B, C, H, W = img.shape
    Hs, Ws = H // 2, W // 2
    Cf = w_feat.shape[1]
    P = _P
    BB = _BB

    img = img.astype(jnp.float32)

    t = jnp.linspace(0.0, 2.0 * jnp.pi, P, endpoint=False)
    ux = (0.5 + 0.45 * jnp.cos(t)).reshape(1, P).astype(jnp.float32)
    uy = (0.5 + 0.45 * jnp.sin(t)).reshape(1, P).astype(jnp.float32)

    wp = jnp.concatenate([w_poly.reshape(3),
                          b_poly.reshape(1)]).astype(jnp.float32)
    wf = w_feat.astype(jnp.float32)
    bf = b_feat.reshape(1, Cf).astype(jnp.float32)
    wc = w_cls.astype(jnp.float32)
    bc = b_cls.reshape(1, 2).astype(jnp.float32)
    wg = w_gcn.astype(jnp.float32)
    bg = b_gcn.reshape(1, 2).astype(jnp.float32)

    outs = pl.pallas_call(
        _fused_kernel,
        out_shape=(
            jax.ShapeDtypeStruct((B, 1, H, W), jnp.float32),
            jax.ShapeDtypeStruct((B, 2), jnp.float32),
            jax.ShapeDtypeStruct((B, P, 2), jnp.float32),
            jax.ShapeDtypeStruct((B, P, 2), jnp.int32),
            jax.ShapeDtypeStruct((B, P, 2), jnp.float32),
            jax.ShapeDtypeStruct((B, P, 2), jnp.int32),
        ),
        grid=(B // BB,),
        in_specs=[
            pl.BlockSpec((1, P), lambda b: (0, 0)),
            pl.BlockSpec((1, P), lambda b: (0, 0)),
            pl.BlockSpec((BB, 4), lambda b: (b, 0)),
            pl.BlockSpec(memory_space=pltpu.MemorySpace.SMEM),
            pl.BlockSpec((BB, C, H, W), lambda b: (b, 0, 0, 0)),
            pl.BlockSpec((C, Cf), lambda b: (0, 0)),
            pl.BlockSpec((1, Cf), lambda b: (0, 0)),
            pl.BlockSpec((C, 2), lambda b: (0, 0)),
            pl.BlockSpec((1, 2), lambda b: (0, 0)),
            pl.BlockSpec((Cf, 2), lambda b: (0, 0)),
            pl.BlockSpec((1, 2), lambda b: (0, 0)),
        ],
        out_specs=(
            pl.BlockSpec((BB, 1, H, W), lambda b: (b, 0, 0, 0)),
            pl.BlockSpec((BB, 2), lambda b: (b, 0)),
            pl.BlockSpec((BB, P, 2), lambda b: (b, 0, 0)),
            pl.BlockSpec((BB, P, 2), lambda b: (b, 0, 0)),
            pl.BlockSpec((BB, P, 2), lambda b: (b, 0, 0)),
            pl.BlockSpec((BB, P, 2), lambda b: (b, 0, 0)),
        ),
        compiler_params=pltpu.CompilerParams(dimension_semantics=("parallel",)),
    )(ux, uy, bbox, wp, img, wf, bf, wc, bc, wg, bg)

    poly_logits, class_prob, pred_polys, original_hull, binary_hull, feature_hull = outs
    return (pred_polys, original_hull, binary_hull, feature_hull,
            poly_logits, class_prob)


# speedup vs baseline: 1.3242x; 1.3242x over previous
import functools

import jax
import jax.numpy as jnp
from jax.experimental import pallas as pl
from jax.experimental.pallas import tpu as pltpu

_BB = 48
_P = 8


def _fused_kernel(ux_ref, uy_ref, bbox_ref, wp_ref, img_ref, wf_ref, bf_ref,
                  wc_ref, bc_ref, wg_ref, bg_ref,
                  poly_ref, cls_ref, pred_ref, orig_ref, binh_ref, feat_ref,
                  *, W):
    BB, C, HW = img_ref.shape
    P = ux_ref.shape[1]
    Hs, Ws = (HW // W) // 2, W // 2

    x = img_ref[...]

    poly_ref[...] = (wp_ref[0] * x[:, 0:1, :] + wp_ref[1] * x[:, 1:2, :]
                     + wp_ref[2] * x[:, 2:3, :] + wp_ref[3])

    pooled = jnp.mean(x, axis=2)
    logits = (pooled[:, 0:1] * wc_ref[0:1, :]
              + pooled[:, 1:2] * wc_ref[1:2, :]
              + pooled[:, 2:3] * wc_ref[2:3, :] + bc_ref[...])
    cls_ref[...] = jax.nn.sigmoid(logits)

    bw = bbox_ref[:, 2:3]
    bh = bbox_ref[:, 3:4]
    hx = jnp.floor(ux_ref[...] * bw)
    hy = jnp.floor(uy_ref[...] * bh)
    wg_ = jnp.maximum(bw, 1e-6)
    hg_ = jnp.maximum(bh, 1e-6)
    orig_ref[...] = jnp.concatenate(
        [hy[:, :, None], hx[:, :, None]], axis=-1).astype(jnp.int32)
    by = hy / hg_
    bx = hx / wg_
    binh = jnp.concatenate([by[:, :, None], bx[:, :, None]], axis=-1)
    binh_ref[...] = binh
    fy = jnp.floor(hy * Hs / hg_).astype(jnp.int32)
    fx = jnp.floor(hx * Ws / wg_).astype(jnp.int32)
    feat_ref[...] = jnp.concatenate(
        [fy[:, :, None], fx[:, :, None]], axis=-1)

    px = 2 * jnp.clip(fy, 0, Hs - 1) * W + 2 * jnp.clip(fx, 0, Ws - 1)
    iota = jax.lax.broadcasted_iota(jnp.int32, (BB, P, HW), 2)
    sel = (iota == px[:, :, None]).astype(jnp.float32)
    g0 = jnp.sum(sel * x[:, 0:1, :], axis=2)
    g1 = jnp.sum(sel * x[:, 1:2, :], axis=2)
    g2 = jnp.sum(sel * x[:, 2:3, :], axis=2)

    wcomb = jnp.dot(wf_ref[...], wg_ref[...],
                    preferred_element_type=jnp.float32)
    const = jnp.dot(bf_ref[...], wg_ref[...],
                    preferred_element_type=jnp.float32) + bg_ref[...]
    offs = (g0[:, :, None] * wcomb[0:1, :]
            + g1[:, :, None] * wcomb[1:2, :]
            + g2[:, :, None] * wcomb[2:3, :] + const)
    pred_ref[...] = binh + offs


def kernel(img, bbox, w_poly, b_poly, w_feat, b_feat, w_cls, b_cls, w_gcn, b_gcn):
    B, C, H, W = img.shape
    HW = H * W
    Cf = w_feat.shape[1]
    P = _P
    BB = _BB if B % _BB == 0 else min(B, 8)

    img_flat = img.reshape(B, C, HW).astype(jnp.float32)

    t = jnp.linspace(0.0, 2.0 * jnp.pi, P, endpoint=False)
    ux = (0.5 + 0.45 * jnp.cos(t)).reshape(1, P).astype(jnp.float32)
    uy = (0.5 + 0.45 * jnp.sin(t)).reshape(1, P).astype(jnp.float32)

    wp = jnp.concatenate([w_poly.reshape(3),
                          b_poly.reshape(1)]).astype(jnp.float32)
    wf = w_feat.astype(jnp.float32)
    bf = b_feat.reshape(1, Cf).astype(jnp.float32)
    wc = w_cls.astype(jnp.float32)
    bc = b_cls.reshape(1, 2).astype(jnp.float32)
    wg = w_gcn.astype(jnp.float32)
    bg = b_gcn.reshape(1, 2).astype(jnp.float32)

    outs = pl.pallas_call(
        functools.partial(_fused_kernel, W=W),
        out_shape=(
            jax.ShapeDtypeStruct((B, 1, HW), jnp.float32),
            jax.ShapeDtypeStruct((B, 2), jnp.float32),
            jax.ShapeDtypeStruct((B, P, 2), jnp.float32),
            jax.ShapeDtypeStruct((B, P, 2), jnp.int32),
            jax.ShapeDtypeStruct((B, P, 2), jnp.float32),
            jax.ShapeDtypeStruct((B, P, 2), jnp.int32),
        ),
        grid=(B // BB,),
        in_specs=[
            pl.BlockSpec((1, P), lambda b: (0, 0)),
            pl.BlockSpec((1, P), lambda b: (0, 0)),
            pl.BlockSpec((BB, 4), lambda b: (b, 0)),
            pl.BlockSpec(memory_space=pltpu.MemorySpace.SMEM),
            pl.BlockSpec((BB, C, HW), lambda b: (b, 0, 0)),
            pl.BlockSpec((C, Cf), lambda b: (0, 0)),
            pl.BlockSpec((1, Cf), lambda b: (0, 0)),
            pl.BlockSpec((C, 2), lambda b: (0, 0)),
            pl.BlockSpec((1, 2), lambda b: (0, 0)),
            pl.BlockSpec((Cf, 2), lambda b: (0, 0)),
            pl.BlockSpec((1, 2), lambda b: (0, 0)),
        ],
        out_specs=(
            pl.BlockSpec((BB, 1, HW), lambda b: (b, 0, 0)),
            pl.BlockSpec((BB, 2), lambda b: (b, 0)),
            pl.BlockSpec((BB, P, 2), lambda b: (b, 0, 0)),
            pl.BlockSpec((BB, P, 2), lambda b: (b, 0, 0)),
            pl.BlockSpec((BB, P, 2), lambda b: (b, 0, 0)),
            pl.BlockSpec((BB, P, 2), lambda b: (b, 0, 0)),
        ),
        compiler_params=pltpu.CompilerParams(dimension_semantics=("parallel",)),
    )(ux, uy, bbox, wp, img_flat, wf, bf, wc, bc, wg, bg)

    poly_out, class_prob, pred_polys, original_hull, binary_hull, feature_hull = outs
    poly_logits = poly_out.reshape(B, 1, H, W)
    return (pred_polys, original_hull, binary_hull, feature_hull,
            poly_logits, class_prob)


# speedup vs baseline: 2.8617x; 2.1611x over previous
import functools

import jax
import jax.numpy as jnp
from jax.experimental import pallas as pl
from jax.experimental.pallas import tpu as pltpu

_P = 8


def _hull_geometry(ux, uy, bbox_t, Hs, Ws):
    bw = bbox_t[2:3, :]
    bh = bbox_t[3:4, :]
    hx = jnp.floor(ux * bw)
    hy = jnp.floor(uy * bh)
    wg_ = jnp.maximum(bw, 1e-6)
    hg_ = jnp.maximum(bh, 1e-6)
    fy = jnp.floor(hy * Hs / hg_).astype(jnp.int32)
    fx = jnp.floor(hx * Ws / wg_).astype(jnp.int32)
    return hy, hx, wg_, hg_, fy, fx


def _enc_kernel(ux_ref, uy_ref, bbox_ref, wp_ref, img_ref,
                poly_ref, pool_ref, gpart_ref, *, HB):
    C, _, W, B = img_ref.shape
    P = ux_ref.shape[0]
    Hs, Ws = None, None
    n_blocks = pl.num_programs(0)
    H = HB * n_blocks
    Hs, Ws = H // 2, W // 2
    s = pl.program_id(0)

    x = img_ref[...]

    poly_ref[...] = (wp_ref[0] * x[0] + wp_ref[1] * x[1]
                     + wp_ref[2] * x[2] + wp_ref[3])

    pool_ref[...] = jnp.sum(x, axis=(1, 2))[None]

    _, _, _, _, fy, fx = _hull_geometry(
        ux_ref[...], uy_ref[...], bbox_ref[...], Hs, Ws)
    rloc = 2 * jnp.clip(fy, 0, Hs - 1) - HB * s
    cloc = 2 * jnp.clip(fx, 0, Ws - 1)
    riota = jax.lax.broadcasted_iota(jnp.int32, (P, HB, W, B), 1)
    ciota = jax.lax.broadcasted_iota(jnp.int32, (P, HB, W, B), 2)
    sel = ((riota == rloc[:, None, None, :])
           & (ciota == cloc[:, None, None, :])).astype(jnp.float32)
    g0 = jnp.sum(sel * x[0][None], axis=(1, 2))
    g1 = jnp.sum(sel * x[1][None], axis=(1, 2))
    g2 = jnp.sum(sel * x[2][None], axis=(1, 2))
    gpart_ref[...] = jnp.stack([g0, g1, g2], axis=0)[None]


def _head_kernel(ux_ref, uy_ref, bbox_ref, pool_ref, gpart_ref, wf_ref,
                 bf_ref, wct_ref, bct_ref, wg_ref, bg_ref,
                 cls_ref, pred_ref, orig_ref, binh_ref, feat_ref,
                 *, Hs, Ws, HW):
    hy, hx, wg_, hg_, fy, fx = _hull_geometry(
        ux_ref[...], uy_ref[...], bbox_ref[...], Hs, Ws)

    orig_ref[...] = jnp.concatenate(
        [hy[:, None, :], hx[:, None, :]], axis=1).astype(jnp.int32)
    by = hy / hg_
    bx = hx / wg_
    binh = jnp.concatenate([by[:, None, :], bx[:, None, :]], axis=1)
    binh_ref[...] = binh
    feat_ref[...] = jnp.concatenate(
        [fy[:, None, :], fx[:, None, :]], axis=1)

    pooled = jnp.sum(pool_ref[...], axis=0) * (1.0 / HW)
    logits = (pooled[0:1, :] * wct_ref[:, 0:1]
              + pooled[1:2, :] * wct_ref[:, 1:2]
              + pooled[2:3, :] * wct_ref[:, 2:3] + bct_ref[...])
    cls_ref[...] = jax.nn.sigmoid(logits)

    g = jnp.sum(gpart_ref[...], axis=0)
    wcomb = jnp.dot(wf_ref[...], wg_ref[...],
                    preferred_element_type=jnp.float32)
    const = jnp.dot(bf_ref[...], wg_ref[...],
                    preferred_element_type=jnp.float32) + bg_ref[...]
    offs = (g[0][:, None, :] * wcomb[0:1, :, None]
            + g[1][:, None, :] * wcomb[1:2, :, None]
            + g[2][:, None, :] * wcomb[2:3, :, None]
            + const[:, :, None])
    pred_ref[...] = binh + offs


def kernel(img, bbox, w_poly, b_poly, w_feat, b_feat, w_cls, b_cls, w_gcn, b_gcn):
    B, C, H, W = img.shape
    HW = H * W
    Hs, Ws = H // 2, W // 2
    Cf = w_feat.shape[1]
    P = _P
    HB = 8 if H % 8 == 0 else H
    NH = H // HB

    img_t = jnp.transpose(img.astype(jnp.float32), (1, 2, 3, 0))
    bbox_t = jnp.transpose(bbox.astype(jnp.float32), (1, 0))

    t = jnp.linspace(0.0, 2.0 * jnp.pi, P, endpoint=False)
    ux = (0.5 + 0.45 * jnp.cos(t)).reshape(P, 1).astype(jnp.float32)
    uy = (0.5 + 0.45 * jnp.sin(t)).reshape(P, 1).astype(jnp.float32)

    wp = jnp.concatenate([w_poly.reshape(3),
                          b_poly.reshape(1)]).astype(jnp.float32)
    wf = w_feat.astype(jnp.float32)
    bf = b_feat.reshape(1, Cf).astype(jnp.float32)
    wct = jnp.transpose(w_cls.astype(jnp.float32), (1, 0))
    bct = b_cls.reshape(2, 1).astype(jnp.float32)
    wg = w_gcn.astype(jnp.float32)
    bg = b_gcn.reshape(1, 2).astype(jnp.float32)

    poly_t, pool_parts, g_parts = pl.pallas_call(
        functools.partial(_enc_kernel, HB=HB),
        out_shape=(
            jax.ShapeDtypeStruct((H, W, B), jnp.float32),
            jax.ShapeDtypeStruct((NH, 3, B), jnp.float32),
            jax.ShapeDtypeStruct((NH, 3, P, B), jnp.float32),
        ),
        grid=(NH,),
        in_specs=[
            pl.BlockSpec((P, 1), lambda s: (0, 0)),
            pl.BlockSpec((P, 1), lambda s: (0, 0)),
            pl.BlockSpec((4, B), lambda s: (0, 0)),
            pl.BlockSpec(memory_space=pltpu.MemorySpace.SMEM),
            pl.BlockSpec((C, HB, W, B), lambda s: (0, s, 0, 0)),
        ],
        out_specs=(
            pl.BlockSpec((HB, W, B), lambda s: (s, 0, 0)),
            pl.BlockSpec((1, 3, B), lambda s: (s, 0, 0)),
            pl.BlockSpec((1, 3, P, B), lambda s: (s, 0, 0, 0)),
        ),
        compiler_params=pltpu.CompilerParams(dimension_semantics=("parallel",)),
    )(ux, uy, bbox_t, wp, img_t)

    cls_t, pred_t, orig_t, binh_t, feat_t = pl.pallas_call(
        functools.partial(_head_kernel, Hs=Hs, Ws=Ws, HW=HW),
        out_shape=(
            jax.ShapeDtypeStruct((2, B), jnp.float32),
            jax.ShapeDtypeStruct((P, 2, B), jnp.float32),
            jax.ShapeDtypeStruct((P, 2, B), jnp.int32),
            jax.ShapeDtypeStruct((P, 2, B), jnp.float32),
            jax.ShapeDtypeStruct((P, 2, B), jnp.int32),
        ),
        grid=(1,),
        in_specs=[
            pl.BlockSpec((P, 1), lambda s: (0, 0)),
            pl.BlockSpec((P, 1), lambda s: (0, 0)),
            pl.BlockSpec((4, B), lambda s: (0, 0)),
            pl.BlockSpec((NH, 3, B), lambda s: (0, 0, 0)),
            pl.BlockSpec((NH, 3, P, B), lambda s: (0, 0, 0, 0)),
            pl.BlockSpec((C, Cf), lambda s: (0, 0)),
            pl.BlockSpec((1, Cf), lambda s: (0, 0)),
            pl.BlockSpec((2, 3), lambda s: (0, 0)),
            pl.BlockSpec((2, 1), lambda s: (0, 0)),
            pl.BlockSpec((Cf, 2), lambda s: (0, 0)),
            pl.BlockSpec((1, 2), lambda s: (0, 0)),
        ],
        out_specs=(
            pl.BlockSpec((2, B), lambda s: (0, 0)),
            pl.BlockSpec((P, 2, B), lambda s: (0, 0, 0)),
            pl.BlockSpec((P, 2, B), lambda s: (0, 0, 0)),
            pl.BlockSpec((P, 2, B), lambda s: (0, 0, 0)),
            pl.BlockSpec((P, 2, B), lambda s: (0, 0, 0)),
        ),
        compiler_params=pltpu.CompilerParams(dimension_semantics=("arbitrary",)),
    )(ux, uy, bbox_t, pool_parts, g_parts, wf, bf, wct, bct, wg, bg)

    pred_polys = jnp.transpose(pred_t, (2, 0, 1))
    original_hull = jnp.transpose(orig_t, (2, 0, 1))
    binary_hull = jnp.transpose(binh_t, (2, 0, 1))
    feature_hull = jnp.transpose(feat_t, (2, 0, 1))
    poly_logits = jnp.transpose(poly_t, (2, 0, 1)).reshape(B, 1, H, W)
    class_prob = jnp.transpose(cls_t, (1, 0))
    return (pred_polys, original_hull, binary_hull, feature_hull,
            poly_logits, class_prob)


# speedup vs baseline: 4.0969x; 1.4317x over previous
import functools

import jax
import jax.numpy as jnp
from jax.experimental import pallas as pl
from jax.experimental.pallas import tpu as pltpu

_P = 8


def _hull_geometry(ux, uy, bbox_t, Hs, Ws):
    bw = bbox_t[2:3, :]
    bh = bbox_t[3:4, :]
    hx = jnp.floor(ux * bw)
    hy = jnp.floor(uy * bh)
    wg_ = jnp.maximum(bw, 1e-6)
    hg_ = jnp.maximum(bh, 1e-6)
    fy = jnp.floor(hy * Hs / hg_).astype(jnp.int32)
    fx = jnp.floor(hx * Ws / wg_).astype(jnp.int32)
    return hy, hx, wg_, hg_, fy, fx


def _enc_kernel(ux_ref, uy_ref, bbox_ref, wp_ref, img_ref, wf_ref, wg_ref,
                poly_ref, pool_ref, gpart_ref, *, HB):
    C, _, W, B = img_ref.shape
    P = ux_ref.shape[0]
    n_blocks = pl.num_programs(0)
    H = HB * n_blocks
    Hs, Ws = H // 2, W // 2
    s = pl.program_id(0)

    x = img_ref[...]

    poly_ref[...] = (wp_ref[0] * x[0] + wp_ref[1] * x[1]
                     + wp_ref[2] * x[2] + wp_ref[3])

    pool_ref[...] = jnp.sum(x, axis=(1, 2))[None]

    _, _, _, _, fy, fx = _hull_geometry(
        ux_ref[...], uy_ref[...], bbox_ref[...], Hs, Ws)
    rloc = 2 * jnp.clip(fy, 0, Hs - 1) - HB * s
    cloc = 2 * jnp.clip(fx, 0, Ws - 1)
    floc = rloc * W + cloc
    wcomb = jnp.dot(wf_ref[...], wg_ref[...],
                    preferred_element_type=jnp.float32)
    x0 = x[0].reshape(HB * W, B)
    x1 = x[1].reshape(HB * W, B)
    x2 = x[2].reshape(HB * W, B)
    xa = wcomb[0, 0] * x0 + wcomb[1, 0] * x1 + wcomb[2, 0] * x2
    xb = wcomb[0, 1] * x0 + wcomb[1, 1] * x1 + wcomb[2, 1] * x2
    fiota = jax.lax.broadcasted_iota(jnp.int32, (P, HB * W, B), 1)
    sel = fiota == floc[:, None, :]
    ga = jnp.sum(jnp.where(sel, xa[None], 0.0), axis=1)
    gb = jnp.sum(jnp.where(sel, xb[None], 0.0), axis=1)
    gpart_ref[...] = jnp.stack([ga, gb], axis=0)[None]


def _head_kernel(ux_ref, uy_ref, bbox_ref, pool_ref, gpart_ref, wf_ref,
                 bf_ref, wct_ref, bct_ref, wg_ref, bg_ref,
                 cls_ref, pred_ref, orig_ref, binh_ref, feat_ref,
                 *, Hs, Ws, HW):
    hy, hx, wg_, hg_, fy, fx = _hull_geometry(
        ux_ref[...], uy_ref[...], bbox_ref[...], Hs, Ws)

    orig_ref[...] = jnp.concatenate(
        [hy[:, None, :], hx[:, None, :]], axis=1).astype(jnp.int32)
    by = hy / hg_
    bx = hx / wg_
    binh = jnp.concatenate([by[:, None, :], bx[:, None, :]], axis=1)
    binh_ref[...] = binh
    feat_ref[...] = jnp.concatenate(
        [fy[:, None, :], fx[:, None, :]], axis=1)

    pooled = jnp.sum(pool_ref[...], axis=0) * (1.0 / HW)
    logits = (pooled[0:1, :] * wct_ref[:, 0:1]
              + pooled[1:2, :] * wct_ref[:, 1:2]
              + pooled[2:3, :] * wct_ref[:, 2:3] + bct_ref[...])
    cls_ref[...] = jax.nn.sigmoid(logits)

    g = jnp.sum(gpart_ref[...], axis=0)
    const = jnp.dot(bf_ref[...], wg_ref[...],
                    preferred_element_type=jnp.float32) + bg_ref[...]
    offs = jnp.concatenate(
        [(g[0] + const[0, 0])[:, None, :],
         (g[1] + const[0, 1])[:, None, :]], axis=1)
    pred_ref[...] = binh + offs


def kernel(img, bbox, w_poly, b_poly, w_feat, b_feat, w_cls, b_cls, w_gcn, b_gcn):
    B, C, H, W = img.shape
    HW = H * W
    Hs, Ws = H // 2, W // 2
    Cf = w_feat.shape[1]
    P = _P
    HB = 8 if H % 8 == 0 else H
    NH = H // HB

    img_t = jnp.transpose(img.astype(jnp.float32), (1, 2, 3, 0))
    bbox_t = jnp.transpose(bbox.astype(jnp.float32), (1, 0))

    t = jnp.linspace(0.0, 2.0 * jnp.pi, P, endpoint=False)
    ux = (0.5 + 0.45 * jnp.cos(t)).reshape(P, 1).astype(jnp.float32)
    uy = (0.5 + 0.45 * jnp.sin(t)).reshape(P, 1).astype(jnp.float32)

    wp = jnp.concatenate([w_poly.reshape(3),
                          b_poly.reshape(1)]).astype(jnp.float32)
    wf = w_feat.astype(jnp.float32)
    bf = b_feat.reshape(1, Cf).astype(jnp.float32)
    wct = jnp.transpose(w_cls.astype(jnp.float32), (1, 0))
    bct = b_cls.reshape(2, 1).astype(jnp.float32)
    wg = w_gcn.astype(jnp.float32)
    bg = b_gcn.reshape(1, 2).astype(jnp.float32)

    poly_t, pool_parts, g_parts = pl.pallas_call(
        functools.partial(_enc_kernel, HB=HB),
        out_shape=(
            jax.ShapeDtypeStruct((H, W, B), jnp.float32),
            jax.ShapeDtypeStruct((NH, 3, B), jnp.float32),
            jax.ShapeDtypeStruct((NH, 2, P, B), jnp.float32),
        ),
        grid=(NH,),
        in_specs=[
            pl.BlockSpec((P, 1), lambda s: (0, 0)),
            pl.BlockSpec((P, 1), lambda s: (0, 0)),
            pl.BlockSpec((4, B), lambda s: (0, 0)),
            pl.BlockSpec(memory_space=pltpu.MemorySpace.SMEM),
            pl.BlockSpec((C, HB, W, B), lambda s: (0, s, 0, 0)),
            pl.BlockSpec((C, Cf), lambda s: (0, 0)),
            pl.BlockSpec((Cf, 2), lambda s: (0, 0)),
        ],
        out_specs=(
            pl.BlockSpec((HB, W, B), lambda s: (s, 0, 0)),
            pl.BlockSpec((1, 3, B), lambda s: (s, 0, 0)),
            pl.BlockSpec((1, 2, P, B), lambda s: (s, 0, 0, 0)),
        ),
        compiler_params=pltpu.CompilerParams(dimension_semantics=("parallel",)),
    )(ux, uy, bbox_t, wp, img_t, wf, wg)

    cls_t, pred_t, orig_t, binh_t, feat_t = pl.pallas_call(
        functools.partial(_head_kernel, Hs=Hs, Ws=Ws, HW=HW),
        out_shape=(
            jax.ShapeDtypeStruct((2, B), jnp.float32),
            jax.ShapeDtypeStruct((P, 2, B), jnp.float32),
            jax.ShapeDtypeStruct((P, 2, B), jnp.int32),
            jax.ShapeDtypeStruct((P, 2, B), jnp.float32),
            jax.ShapeDtypeStruct((P, 2, B), jnp.int32),
        ),
        grid=(1,),
        in_specs=[
            pl.BlockSpec((P, 1), lambda s: (0, 0)),
            pl.BlockSpec((P, 1), lambda s: (0, 0)),
            pl.BlockSpec((4, B), lambda s: (0, 0)),
            pl.BlockSpec((NH, 3, B), lambda s: (0, 0, 0)),
            pl.BlockSpec((NH, 2, P, B), lambda s: (0, 0, 0, 0)),
            pl.BlockSpec((C, Cf), lambda s: (0, 0)),
            pl.BlockSpec((1, Cf), lambda s: (0, 0)),
            pl.BlockSpec((2, 3), lambda s: (0, 0)),
            pl.BlockSpec((2, 1), lambda s: (0, 0)),
            pl.BlockSpec((Cf, 2), lambda s: (0, 0)),
            pl.BlockSpec((1, 2), lambda s: (0, 0)),
        ],
        out_specs=(
            pl.BlockSpec((2, B), lambda s: (0, 0)),
            pl.BlockSpec((P, 2, B), lambda s: (0, 0, 0)),
            pl.BlockSpec((P, 2, B), lambda s: (0, 0, 0)),
            pl.BlockSpec((P, 2, B), lambda s: (0, 0, 0)),
            pl.BlockSpec((P, 2, B), lambda s: (0, 0, 0)),
        ),
        compiler_params=pltpu.CompilerParams(dimension_semantics=("arbitrary",)),
    )(ux, uy, bbox_t, pool_parts, g_parts, wf, bf, wct, bct, wg, bg)

    pred_polys = jnp.transpose(pred_t, (2, 0, 1))
    original_hull = jnp.transpose(orig_t, (2, 0, 1))
    binary_hull = jnp.transpose(binh_t, (2, 0, 1))
    feature_hull = jnp.transpose(feat_t, (2, 0, 1))
    poly_logits = jnp.transpose(poly_t, (2, 0, 1)).reshape(B, 1, H, W)
    class_prob = jnp.transpose(cls_t, (1, 0))
    return (pred_polys, original_hull, binary_hull, feature_hull,
            poly_logits, class_prob)


# speedup vs baseline: 5.4556x; 1.3316x over previous
import functools

import jax
import jax.numpy as jnp
from jax.experimental import pallas as pl
from jax.experimental.pallas import tpu as pltpu

_P = 8


def _hull_geometry(ux, uy, bbox_t, Hs, Ws):
    bw = bbox_t[2:3, :]
    bh = bbox_t[3:4, :]
    hx = jnp.floor(ux * bw)
    hy = jnp.floor(uy * bh)
    wg_ = jnp.maximum(bw, 1e-6)
    hg_ = jnp.maximum(bh, 1e-6)
    fy = jnp.floor(hy * Hs / hg_).astype(jnp.int32)
    fx = jnp.floor(hx * Ws / wg_).astype(jnp.int32)
    return hy, hx, wg_, hg_, fy, fx


def _enc_kernel(ux_ref, uy_ref, bbox_ref, wp_ref, img_ref, wf_ref, wg_ref,
                poly_ref, pool_ref, gpart_ref, *, HB):
    C, _, W, B = img_ref.shape
    P = ux_ref.shape[0]
    n_blocks = pl.num_programs(0)
    H = HB * n_blocks
    Hs, Ws = H // 2, W // 2
    s = pl.program_id(0)

    x = img_ref[...]

    poly_ref[...] = (wp_ref[0] * x[0] + wp_ref[1] * x[1]
                     + wp_ref[2] * x[2] + wp_ref[3])

    pool_ref[...] = jnp.sum(x, axis=(1, 2))[None]

    _, _, _, _, fy, fx = _hull_geometry(
        ux_ref[...], uy_ref[...], bbox_ref[...], Hs, Ws)
    HBh = HB // 2
    Q = HBh * Ws
    rloc = jnp.clip(fy, 0, Hs - 1) - HBh * s
    cloc = jnp.clip(fx, 0, Ws - 1)
    floc = rloc * Ws + cloc
    wcomb = jnp.dot(wf_ref[...], wg_ref[...],
                    preferred_element_type=jnp.float32)

    def subsample(xc):
        xe = xc.reshape(HBh, 2, W, B)[:, 0]
        return xe.reshape(HBh * W, B).reshape(Q, 2, B)[:, 0]

    x0 = subsample(x[0])
    x1 = subsample(x[1])
    x2 = subsample(x[2])
    xa = wcomb[0, 0] * x0 + wcomb[1, 0] * x1 + wcomb[2, 0] * x2
    xb = wcomb[0, 1] * x0 + wcomb[1, 1] * x1 + wcomb[2, 1] * x2
    fiota = jax.lax.broadcasted_iota(jnp.int32, (P, Q, B), 1)
    sel = fiota == floc[:, None, :]
    ga = jnp.sum(jnp.where(sel, xa[None], 0.0), axis=1)
    gb = jnp.sum(jnp.where(sel, xb[None], 0.0), axis=1)
    gpart_ref[...] = jnp.stack([ga, gb], axis=0)[None]


def _head_kernel(ux_ref, uy_ref, bbox_ref, pool_ref, gpart_ref, wf_ref,
                 bf_ref, wct_ref, bct_ref, wg_ref, bg_ref,
                 cls_ref, pred_ref, orig_ref, binh_ref, feat_ref,
                 *, Hs, Ws, HW):
    hy, hx, wg_, hg_, fy, fx = _hull_geometry(
        ux_ref[...], uy_ref[...], bbox_ref[...], Hs, Ws)

    orig_ref[...] = jnp.concatenate(
        [hy[:, None, :], hx[:, None, :]], axis=1).astype(jnp.int32)
    by = hy / hg_
    bx = hx / wg_
    binh = jnp.concatenate([by[:, None, :], bx[:, None, :]], axis=1)
    binh_ref[...] = binh
    feat_ref[...] = jnp.concatenate(
        [fy[:, None, :], fx[:, None, :]], axis=1)

    pooled = jnp.sum(pool_ref[...], axis=0) * (1.0 / HW)
    logits = (pooled[0:1, :] * wct_ref[:, 0:1]
              + pooled[1:2, :] * wct_ref[:, 1:2]
              + pooled[2:3, :] * wct_ref[:, 2:3] + bct_ref[...])
    cls_ref[...] = jax.nn.sigmoid(logits)

    g = jnp.sum(gpart_ref[...], axis=0)
    const = jnp.dot(bf_ref[...], wg_ref[...],
                    preferred_element_type=jnp.float32) + bg_ref[...]
    offs = jnp.concatenate(
        [(g[0] + const[0, 0])[:, None, :],
         (g[1] + const[0, 1])[:, None, :]], axis=1)
    pred_ref[...] = binh + offs


def kernel(img, bbox, w_poly, b_poly, w_feat, b_feat, w_cls, b_cls, w_gcn, b_gcn):
    B, C, H, W = img.shape
    HW = H * W
    Hs, Ws = H // 2, W // 2
    Cf = w_feat.shape[1]
    P = _P
    HB = 8 if H % 8 == 0 else H
    NH = H // HB

    img_t = jnp.transpose(img.astype(jnp.float32), (1, 2, 3, 0))
    bbox_t = jnp.transpose(bbox.astype(jnp.float32), (1, 0))

    t = jnp.linspace(0.0, 2.0 * jnp.pi, P, endpoint=False)
    ux = (0.5 + 0.45 * jnp.cos(t)).reshape(P, 1).astype(jnp.float32)
    uy = (0.5 + 0.45 * jnp.sin(t)).reshape(P, 1).astype(jnp.float32)

    wp = jnp.concatenate([w_poly.reshape(3),
                          b_poly.reshape(1)]).astype(jnp.float32)
    wf = w_feat.astype(jnp.float32)
    bf = b_feat.reshape(1, Cf).astype(jnp.float32)
    wct = jnp.transpose(w_cls.astype(jnp.float32), (1, 0))
    bct = b_cls.reshape(2, 1).astype(jnp.float32)
    wg = w_gcn.astype(jnp.float32)
    bg = b_gcn.reshape(1, 2).astype(jnp.float32)

    poly_t, pool_parts, g_parts = pl.pallas_call(
        functools.partial(_enc_kernel, HB=HB),
        out_shape=(
            jax.ShapeDtypeStruct((H, W, B), jnp.float32),
            jax.ShapeDtypeStruct((NH, 3, B), jnp.float32),
            jax.ShapeDtypeStruct((NH, 2, P, B), jnp.float32),
        ),
        grid=(NH,),
        in_specs=[
            pl.BlockSpec((P, 1), lambda s: (0, 0)),
            pl.BlockSpec((P, 1), lambda s: (0, 0)),
            pl.BlockSpec((4, B), lambda s: (0, 0)),
            pl.BlockSpec(memory_space=pltpu.MemorySpace.SMEM),
            pl.BlockSpec((C, HB, W, B), lambda s: (0, s, 0, 0)),
            pl.BlockSpec((C, Cf), lambda s: (0, 0)),
            pl.BlockSpec((Cf, 2), lambda s: (0, 0)),
        ],
        out_specs=(
            pl.BlockSpec((HB, W, B), lambda s: (s, 0, 0)),
            pl.BlockSpec((1, 3, B), lambda s: (s, 0, 0)),
            pl.BlockSpec((1, 2, P, B), lambda s: (s, 0, 0, 0)),
        ),
        compiler_params=pltpu.CompilerParams(dimension_semantics=("parallel",)),
    )(ux, uy, bbox_t, wp, img_t, wf, wg)

    cls_t, pred_t, orig_t, binh_t, feat_t = pl.pallas_call(
        functools.partial(_head_kernel, Hs=Hs, Ws=Ws, HW=HW),
        out_shape=(
            jax.ShapeDtypeStruct((2, B), jnp.float32),
            jax.ShapeDtypeStruct((P, 2, B), jnp.float32),
            jax.ShapeDtypeStruct((P, 2, B), jnp.int32),
            jax.ShapeDtypeStruct((P, 2, B), jnp.float32),
            jax.ShapeDtypeStruct((P, 2, B), jnp.int32),
        ),
        grid=(1,),
        in_specs=[
            pl.BlockSpec((P, 1), lambda s: (0, 0)),
            pl.BlockSpec((P, 1), lambda s: (0, 0)),
            pl.BlockSpec((4, B), lambda s: (0, 0)),
            pl.BlockSpec((NH, 3, B), lambda s: (0, 0, 0)),
            pl.BlockSpec((NH, 2, P, B), lambda s: (0, 0, 0, 0)),
            pl.BlockSpec((C, Cf), lambda s: (0, 0)),
            pl.BlockSpec((1, Cf), lambda s: (0, 0)),
            pl.BlockSpec((2, 3), lambda s: (0, 0)),
            pl.BlockSpec((2, 1), lambda s: (0, 0)),
            pl.BlockSpec((Cf, 2), lambda s: (0, 0)),
            pl.BlockSpec((1, 2), lambda s: (0, 0)),
        ],
        out_specs=(
            pl.BlockSpec((2, B), lambda s: (0, 0)),
            pl.BlockSpec((P, 2, B), lambda s: (0, 0, 0)),
            pl.BlockSpec((P, 2, B), lambda s: (0, 0, 0)),
            pl.BlockSpec((P, 2, B), lambda s: (0, 0, 0)),
            pl.BlockSpec((P, 2, B), lambda s: (0, 0, 0)),
        ),
        compiler_params=pltpu.CompilerParams(dimension_semantics=("arbitrary",)),
    )(ux, uy, bbox_t, pool_parts, g_parts, wf, bf, wct, bct, wg, bg)

    pred_polys = jnp.transpose(pred_t, (2, 0, 1))
    original_hull = jnp.transpose(orig_t, (2, 0, 1))
    binary_hull = jnp.transpose(binh_t, (2, 0, 1))
    feature_hull = jnp.transpose(feat_t, (2, 0, 1))
    poly_logits = jnp.transpose(poly_t, (2, 0, 1)).reshape(B, 1, H, W)
    class_prob = jnp.transpose(cls_t, (1, 0))
    return (pred_polys, original_hull, binary_hull, feature_hull,
            poly_logits, class_prob)
